# baseline (speedup 1.0000x reference)
import functools

import jax
import jax.numpy as jnp
from jax import lax
from jax.experimental import pallas as pl
from jax.experimental.pallas import tpu as pltpu
import jax.experimental.pallas.tpu_sc as plsc

_N = 10000
_NP = 10240
_E = 320000
_D = 128
_ALPHA = 0.1
_BETA = 1.0 - _ALPHA

_NC = 2
_NS = 16
_CH = 80
_EW = _E // _NS
_NCH = _EW // _CH
_RW = _NP // _NS
_ZR = 128
_NB = 2

_mesh = plsc.VectorSubcoreMesh(
    core_axis_name="c", subcore_axis_name="s", num_cores=_NC, num_subcores=_NS)


@functools.partial(
    pl.kernel,
    out_type=jax.ShapeDtypeStruct((2, 2, _NP, _D), jnp.float32),
    mesh=_mesh,
    scratch_types=[
        pltpu.VMEM_SHARED((_NP, _D), jnp.float32),
        pltpu.VMEM((_ZR, _D), jnp.float32),
        pltpu.VMEM((_CH, _D), jnp.float32),
        pltpu.VMEM((_CH,), jnp.int32),
    ],
)
def _deg_kernel(edges, hist_out, acc, zbuf, ones_v, idx_b):
    c = lax.axis_index("c")
    s = lax.axis_index("s")
    zero16 = jnp.zeros((16,), jnp.float32)
    one16 = jnp.ones((16,), jnp.float32)

    def _fill_o(i, carry):
        for u in range(_D // 16):
            ones_v[i, pl.ds(u * 16, 16)] = one16
        return carry

    lax.fori_loop(0, _CH, _fill_o, 0)

    for k in (0, 1):
        def _fill_z(i, carry):
            for u in range(_D // 16):
                zbuf[i, pl.ds(u * 16, 16)] = zero16
            return carry

        lax.fori_loop(0, _ZR, _fill_z, 0)
        for q in range(_RW // _ZR):
            pltpu.sync_copy(zbuf, acc.at[pl.ds(s * _RW + q * _ZR, _ZR)])
        plsc.subcore_barrier()

        def _chunk(j, carry):
            off = s * _EW + j * _CH
            pltpu.sync_copy(edges.at[pl.ds((c * 2 + k) * _E + off, _CH)],
                            idx_b)
            pltpu.sync_copy(ones_v, acc.at[idx_b], add=True)
            return carry

        lax.fori_loop(0, _NCH, _chunk, 0)
        plsc.subcore_barrier()

        for q in range(_RW // _ZR):
            r0 = s * _RW + q * _ZR
            pltpu.sync_copy(acc.at[pl.ds(r0, _ZR)], zbuf)
            pltpu.sync_copy(zbuf, hist_out.at[c, k, pl.ds(r0, _ZR)])
        plsc.subcore_barrier()


@functools.partial(
    pl.kernel,
    out_type=jax.ShapeDtypeStruct((2, _NP, _D), jnp.float32),
    mesh=_mesh,
    scratch_types=[
        pltpu.VMEM_SHARED((_NP, _D), jnp.float32),
        [pltpu.VMEM((_CH, _D), jnp.float32)] * _NB,
        [pltpu.VMEM((_CH,), jnp.int32)] * _NB,
        [pltpu.VMEM((_CH,), jnp.int32)] * _NB,
        [pltpu.SemaphoreType.DMA] * _NB,
        [pltpu.SemaphoreType.DMA] * _NB,
    ],
)
def _agg_kernel(xs, edges, agg_out, acc, rows, i_s, i_d, gsem, ssem):
    c = lax.axis_index("c")
    s = lax.axis_index("s")
    zero16 = jnp.zeros((16,), jnp.float32)
    cbase = c * _NP

    def _zrow(i, carry):
        for u in range(_D // 16):
            rows[0][i, pl.ds(u * 16, 16)] = zero16
        return carry

    lax.fori_loop(0, _CH, _zrow, 0)
    for q in range(_RW // _CH):
        pltpu.sync_copy(rows[0], acc.at[pl.ds(s * _RW + q * _CH, _CH)])
    plsc.subcore_barrier()

    def _load_idx(j, t):
        off = s * _EW + j * _CH
        pltpu.sync_copy(edges.at[pl.ds((c * 2 + 0) * _E + off, _CH)], i_s[t])
        pltpu.sync_copy(edges.at[pl.ds((c * 2 + 1) * _E + off, _CH)], i_d[t])
        for u in range(_CH // 16):
            i_s[t][pl.ds(u * 16, 16)] = i_s[t][pl.ds(u * 16, 16)] + cbase

    def _gather(t):
        pltpu.async_copy(xs.at[i_s[t]], rows[t], gsem[t])

    def _gwait(t):
        pltpu.make_async_copy(xs.at[i_s[t]], rows[t], gsem[t]).wait()

    def _scat(t):
        pltpu.async_copy(rows[t], acc.at[i_d[t]], ssem[t], add=True)

    def _swait(t):
        pltpu.make_async_copy(rows[t], acc.at[i_d[t]], ssem[t]).wait()

    for t in range(_NB):
        _load_idx(t, t)
        _gather(t)

    def _body(j, carry):
        base = j * _NB
        for t in range(_NB):
            _gwait(t)
            _scat(t)
        for t in range(_NB):
            _swait(t)
            _load_idx(base + _NB + t, t)
            _gather(t)
        return carry

    lax.fori_loop(0, _NCH // _NB - 1, _body, 0)
    for t in range(_NB):
        _gwait(t)
        _scat(t)
    for t in range(_NB):
        _swait(t)
    plsc.subcore_barrier()

    for q in range(_RW // _CH):
        r0 = s * _RW + q * _CH
        pltpu.sync_copy(acc.at[pl.ds(r0, _CH)], rows[0])
        pltpu.sync_copy(rows[0], agg_out.at[c, pl.ds(r0, _CH)])


_BN = 1000


def _prep_body(h_ref, w_ref, b_ref, hist_ref, z_ref, xs_ref, ns_ref, nd_ref):
    zb = jnp.maximum(
        jnp.dot(h_ref[...], w_ref[...], preferred_element_type=jnp.float32)
        + b_ref[...], 0.0)
    z_ref[...] = zb
    d = hist_ref[...][:, :, :, 0]
    nrm = jnp.where(d > 0, lax.rsqrt(d), 0.0)
    ns = nrm[:, 0, :, None]
    ns_ref[...] = ns
    nd_ref[...] = nrm[:, 1, :, None]
    xs_ref[...] = zb[None, :, :] * ns


def _prep_call(h, W, b2, hist):
    return pl.pallas_call(
        _prep_body,
        grid=(_N // _BN,),
        in_specs=[
            pl.BlockSpec((_BN, _D), lambda i: (i, 0)),
            pl.BlockSpec((_D, _D), lambda i: (0, 0)),
            pl.BlockSpec((1, _D), lambda i: (0, 0)),
            pl.BlockSpec((2, 2, _BN, _D), lambda i: (0, 0, i, 0)),
        ],
        out_specs=[
            pl.BlockSpec((_BN, _D), lambda i: (i, 0)),
            pl.BlockSpec((2, _BN, _D), lambda i: (0, i, 0)),
            pl.BlockSpec((2, _BN, 1), lambda i: (0, i, 0)),
            pl.BlockSpec((2, _BN, 1), lambda i: (0, i, 0)),
        ],
        out_shape=[
            jax.ShapeDtypeStruct((_N, _D), jnp.float32),
            jax.ShapeDtypeStruct((2, _NP, _D), jnp.float32),
            jax.ShapeDtypeStruct((2, _N, 1), jnp.float32),
            jax.ShapeDtypeStruct((2, _N, 1), jnp.float32),
        ],
    )(h, W, b2, hist)


def _upd_body(agg_ref, z_ref, ns_ref, nd_ref, xs_ref):
    ns = ns_ref[0]
    nd = nd_ref[0]
    xs_ref[0] = (_BETA * ns * nd) * agg_ref[0] + (_ALPHA * ns) * z_ref[...]


def _upd_call(agg, z, ns, nd):
    return pl.pallas_call(
        _upd_body,
        grid=(2, _N // _BN),
        in_specs=[
            pl.BlockSpec((1, _BN, _D), lambda p, i: (p, i, 0)),
            pl.BlockSpec((_BN, _D), lambda p, i: (i, 0)),
            pl.BlockSpec((1, _BN, 1), lambda p, i: (p, i, 0)),
            pl.BlockSpec((1, _BN, 1), lambda p, i: (p, i, 0)),
        ],
        out_specs=pl.BlockSpec((1, _BN, _D), lambda p, i: (p, i, 0)),
        out_shape=jax.ShapeDtypeStruct((2, _NP, _D), jnp.float32),
    )(agg, z, ns, nd)


def _fin_body(agg_ref, z_ref, nd_ref, x_ref):
    nd = nd_ref[0]
    x_ref[0] = (_BETA * nd) * agg_ref[0] + _ALPHA * z_ref[...]


def _fin_call(agg, z, nd):
    return pl.pallas_call(
        _fin_body,
        grid=(2, _N // _BN),
        in_specs=[
            pl.BlockSpec((1, _BN, _D), lambda p, i: (p, i, 0)),
            pl.BlockSpec((_BN, _D), lambda p, i: (i, 0)),
            pl.BlockSpec((1, _BN, 1), lambda p, i: (p, i, 0)),
        ],
        out_specs=pl.BlockSpec((1, _BN, _D), lambda p, i: (p, i, 0)),
        out_shape=jax.ShapeDtypeStruct((2, _N, _D), jnp.float32),
    )(agg, z, nd)


def _att_body(x_ref, wa_ref, ba_ref, qa_ref, acc_ref):
    i = pl.program_id(1)

    @pl.when(i == 0)
    def _():
        acc_ref[...] = jnp.zeros_like(acc_ref)

    t = jnp.tanh(
        jnp.dot(x_ref[0], wa_ref[...], preferred_element_type=jnp.float32)
        + ba_ref[...])
    acc_ref[...] += jnp.sum(t * qa_ref[...])


def _att_call(x, Wa, ba2, qa2):
    return pl.pallas_call(
        _att_body,
        grid=(2, _N // _BN),
        in_specs=[
            pl.BlockSpec((1, _BN, _D), lambda p, i: (p, i, 0)),
            pl.BlockSpec((_D, _D), lambda p, i: (0, 0)),
            pl.BlockSpec((1, _D), lambda p, i: (0, 0)),
            pl.BlockSpec((1, _D), lambda p, i: (0, 0)),
        ],
        out_specs=pl.BlockSpec((1, 8, _D), lambda p, i: (p, 0, 0)),
        out_shape=jax.ShapeDtypeStruct((2, 8, _D), jnp.float32),
    )(x, Wa, ba2, qa2)


def _comb_body(x_ref, acc_ref, o_ref):
    w0 = acc_ref[0, 0, 0] * (1.0 / _N)
    w1 = acc_ref[1, 0, 0] * (1.0 / _N)
    m = jnp.maximum(w0, w1)
    e0 = jnp.exp(w0 - m)
    e1 = jnp.exp(w1 - m)
    r = 1.0 / (e0 + e1)
    o_ref[...] = (e0 * r) * x_ref[0] + (e1 * r) * x_ref[1]


def _comb_call(x, wa):
    return pl.pallas_call(
        _comb_body,
        grid=(_N // _BN,),
        in_specs=[
            pl.BlockSpec((2, _BN, _D), lambda i: (0, i, 0)),
            pl.BlockSpec((2, 8, _D), lambda i: (0, 0, 0)),
        ],
        out_specs=pl.BlockSpec((_BN, _D), lambda i: (i, 0)),
        out_shape=jax.ShapeDtypeStruct((_N, _D), jnp.float32),
    )(x, wa)


def kernel(h, edge_index_0, edge_index_1, W_hidden, b_hidden, W_att, b_att,
           q_att):
    edges = jnp.stack([edge_index_0, edge_index_1]).reshape(4 * _E)
    hist = _deg_kernel(edges)
    z, xs, ns, nd = _prep_call(h, W_hidden, b_hidden.reshape(1, _D), hist)
    x = None
    for k in range(3):
        agg = _agg_kernel(xs.reshape(2 * _NP, _D), edges)
        if k < 2:
            xs = _upd_call(agg, z, ns, nd)
        else:
            x = _fin_call(agg, z, nd)
    wa = _att_call(x, W_att, b_att.reshape(1, _D), q_att.reshape(1, _D))
    return _comb_call(x, wa)


# speedup vs baseline: 1.0528x; 1.0528x over previous
import functools

import jax
import jax.numpy as jnp
from jax import lax
from jax.experimental import pallas as pl
from jax.experimental.pallas import tpu as pltpu
import jax.experimental.pallas.tpu_sc as plsc

_N = 10000
_NP = 10240
_E = 320000
_D = 128
_ALPHA = 0.1
_BETA = 1.0 - _ALPHA

_NC = 2
_NS = 16
_CH = 80
_EW = _E // _NS
_NCH = _EW // _CH
_RW = _NP // _NS
_ZR = 128
_NB = 2

_mesh = plsc.VectorSubcoreMesh(
    core_axis_name="c", subcore_axis_name="s", num_cores=_NC, num_subcores=_NS)


@functools.partial(
    pl.kernel,
    out_type=jax.ShapeDtypeStruct((2, _NP, _D), jnp.float32),
    mesh=_mesh,
    scratch_types=[
        pltpu.VMEM_SHARED((_NP, _D), jnp.float32),
        pltpu.VMEM((_ZR, _D), jnp.float32),
        pltpu.VMEM((_CH, _D), jnp.float32),
        pltpu.VMEM((_CH,), jnp.int32),
        pltpu.VMEM((_CH,), jnp.int32),
    ],
)
def _deg_kernel(edges, hist_out, acc, zbuf, ones_s, idx_s, idx_d):
    c = lax.axis_index("c")
    s = lax.axis_index("s")
    zero16 = jnp.zeros((16,), jnp.float32)
    one16 = jnp.ones((16,), jnp.float32)

    def _fill_o(i, carry):
        for u in range(_D // 16):
            ones_s[i, pl.ds(u * 16, 16)] = one16 if u < 4 else zero16
        return carry

    lax.fori_loop(0, _CH, _fill_o, 0)

    def _fill_z(i, carry):
        for u in range(_D // 16):
            zbuf[i, pl.ds(u * 16, 16)] = zero16
        return carry

    lax.fori_loop(0, _ZR, _fill_z, 0)
    for q in range(_RW // _ZR):
        pltpu.sync_copy(zbuf, acc.at[pl.ds(s * _RW + q * _ZR, _ZR)])

    def _fill_d(i, carry):
        for u in range(_D // 16):
            zbuf[i, pl.ds(u * 16, 16)] = zero16 if u < 4 else one16
        return carry

    lax.fori_loop(0, _CH, _fill_d, 0)
    plsc.subcore_barrier()

    def _chunk(j, carry):
        off = s * _EW + j * _CH
        pltpu.sync_copy(edges.at[pl.ds((c * 2 + 0) * _E + off, _CH)], idx_s)
        pltpu.sync_copy(edges.at[pl.ds((c * 2 + 1) * _E + off, _CH)], idx_d)
        pltpu.sync_copy(ones_s, acc.at[idx_s], add=True)
        pltpu.sync_copy(zbuf.at[pl.ds(0, _CH)], acc.at[idx_d], add=True)
        return carry

    lax.fori_loop(0, _NCH, _chunk, 0)
    plsc.subcore_barrier()

    for q in range(_RW // _ZR):
        r0 = s * _RW + q * _ZR
        pltpu.sync_copy(acc.at[pl.ds(r0, _ZR)], zbuf)
        pltpu.sync_copy(zbuf, hist_out.at[c, pl.ds(r0, _ZR)])


@functools.partial(
    pl.kernel,
    out_type=jax.ShapeDtypeStruct((2, _NP, _D), jnp.float32),
    mesh=_mesh,
    scratch_types=[
        pltpu.VMEM_SHARED((_NP, _D), jnp.float32),
        [pltpu.VMEM((_CH, _D), jnp.float32)] * _NB,
        [pltpu.VMEM((_CH,), jnp.int32)] * _NB,
        [pltpu.VMEM((_CH,), jnp.int32)] * _NB,
        [pltpu.SemaphoreType.DMA] * _NB,
    ],
)
def _agg_kernel(xs, edges, agg_out, acc, rows, i_s, i_d, gsem):
    c = lax.axis_index("c")
    s = lax.axis_index("s")
    zero16 = jnp.zeros((16,), jnp.float32)
    cbase = c * _NP

    def _zrow(i, carry):
        for u in range(_D // 16):
            rows[0][i, pl.ds(u * 16, 16)] = zero16
        return carry

    lax.fori_loop(0, _CH, _zrow, 0)
    for q in range(_RW // _CH):
        pltpu.sync_copy(rows[0], acc.at[pl.ds(s * _RW + q * _CH, _CH)])
    plsc.subcore_barrier()

    def _load_idx(j, t):
        off = s * _EW + j * _CH
        pltpu.sync_copy(edges.at[pl.ds((c * 2 + 0) * _E + off, _CH)], i_s[t])
        pltpu.sync_copy(edges.at[pl.ds((c * 2 + 1) * _E + off, _CH)], i_d[t])
        for u in range(_CH // 16):
            i_s[t][pl.ds(u * 16, 16)] = i_s[t][pl.ds(u * 16, 16)] + cbase

    def _gather(t):
        pltpu.async_copy(xs.at[i_s[t]], rows[t], gsem[t])

    def _gwait(t):
        pltpu.make_async_copy(xs.at[i_s[t]], rows[t], gsem[t]).wait()

    def _scat(t):
        pltpu.sync_copy(rows[t], acc.at[i_d[t]], add=True)

    for t in range(_NB):
        _load_idx(t, t)
        _gather(t)

    def _body(j, carry):
        base = j * _NB
        for t in range(_NB):
            _gwait(t)
            _scat(t)
            _load_idx(base + _NB + t, t)
            _gather(t)
        return carry

    lax.fori_loop(0, _NCH // _NB - 1, _body, 0)
    for t in range(_NB):
        _gwait(t)
        _scat(t)
    plsc.subcore_barrier()

    for q in range(_RW // _CH):
        r0 = s * _RW + q * _CH
        pltpu.sync_copy(acc.at[pl.ds(r0, _CH)], rows[0])
        pltpu.sync_copy(rows[0], agg_out.at[c, pl.ds(r0, _CH)])


_BN = 1000


def _prep_body(h_ref, w_ref, b_ref, hist_ref, z_ref, xs_ref, ns_ref, nd_ref):
    zb = jnp.maximum(
        jnp.dot(h_ref[...], w_ref[...], preferred_element_type=jnp.float32)
        + b_ref[...], 0.0)
    z_ref[...] = zb
    h3 = hist_ref[...]
    ds_ = h3[:, :, 0]
    dd_ = h3[:, :, 64]
    ns = jnp.where(ds_ > 0, lax.rsqrt(ds_), 0.0)[:, :, None]
    ns_ref[...] = ns
    nd_ref[...] = jnp.where(dd_ > 0, lax.rsqrt(dd_), 0.0)[:, :, None]
    xs_ref[...] = zb[None, :, :] * ns


def _prep_call(h, W, b2, hist):
    return pl.pallas_call(
        _prep_body,
        grid=(_N // _BN,),
        in_specs=[
            pl.BlockSpec((_BN, _D), lambda i: (i, 0)),
            pl.BlockSpec((_D, _D), lambda i: (0, 0)),
            pl.BlockSpec((1, _D), lambda i: (0, 0)),
            pl.BlockSpec((2, _BN, _D), lambda i: (0, i, 0)),
        ],
        out_specs=[
            pl.BlockSpec((_BN, _D), lambda i: (i, 0)),
            pl.BlockSpec((2, _BN, _D), lambda i: (0, i, 0)),
            pl.BlockSpec((2, _BN, 1), lambda i: (0, i, 0)),
            pl.BlockSpec((2, _BN, 1), lambda i: (0, i, 0)),
        ],
        out_shape=[
            jax.ShapeDtypeStruct((_N, _D), jnp.float32),
            jax.ShapeDtypeStruct((2, _NP, _D), jnp.float32),
            jax.ShapeDtypeStruct((2, _N, 1), jnp.float32),
            jax.ShapeDtypeStruct((2, _N, 1), jnp.float32),
        ],
    )(h, W, b2, hist)


def _upd_body(agg_ref, z_ref, ns_ref, nd_ref, xs_ref):
    ns = ns_ref[0]
    nd = nd_ref[0]
    xs_ref[0] = (_BETA * ns * nd) * agg_ref[0] + (_ALPHA * ns) * z_ref[...]


def _upd_call(agg, z, ns, nd):
    return pl.pallas_call(
        _upd_body,
        grid=(2, _N // _BN),
        in_specs=[
            pl.BlockSpec((1, _BN, _D), lambda p, i: (p, i, 0)),
            pl.BlockSpec((_BN, _D), lambda p, i: (i, 0)),
            pl.BlockSpec((1, _BN, 1), lambda p, i: (p, i, 0)),
            pl.BlockSpec((1, _BN, 1), lambda p, i: (p, i, 0)),
        ],
        out_specs=pl.BlockSpec((1, _BN, _D), lambda p, i: (p, i, 0)),
        out_shape=jax.ShapeDtypeStruct((2, _NP, _D), jnp.float32),
    )(agg, z, ns, nd)


def _fin_body(agg_ref, z_ref, nd_ref, x_ref):
    nd = nd_ref[0]
    x_ref[0] = (_BETA * nd) * agg_ref[0] + _ALPHA * z_ref[...]


def _fin_call(agg, z, nd):
    return pl.pallas_call(
        _fin_body,
        grid=(2, _N // _BN),
        in_specs=[
            pl.BlockSpec((1, _BN, _D), lambda p, i: (p, i, 0)),
            pl.BlockSpec((_BN, _D), lambda p, i: (i, 0)),
            pl.BlockSpec((1, _BN, 1), lambda p, i: (p, i, 0)),
        ],
        out_specs=pl.BlockSpec((1, _BN, _D), lambda p, i: (p, i, 0)),
        out_shape=jax.ShapeDtypeStruct((2, _N, _D), jnp.float32),
    )(agg, z, nd)


def _att_body(x_ref, wa_ref, ba_ref, qa_ref, acc_ref):
    i = pl.program_id(1)

    @pl.when(i == 0)
    def _():
        acc_ref[...] = jnp.zeros_like(acc_ref)

    t = jnp.tanh(
        jnp.dot(x_ref[0], wa_ref[...], preferred_element_type=jnp.float32)
        + ba_ref[...])
    acc_ref[...] += jnp.sum(t * qa_ref[...])


def _att_call(x, Wa, ba2, qa2):
    return pl.pallas_call(
        _att_body,
        grid=(2, _N // _BN),
        in_specs=[
            pl.BlockSpec((1, _BN, _D), lambda p, i: (p, i, 0)),
            pl.BlockSpec((_D, _D), lambda p, i: (0, 0)),
            pl.BlockSpec((1, _D), lambda p, i: (0, 0)),
            pl.BlockSpec((1, _D), lambda p, i: (0, 0)),
        ],
        out_specs=pl.BlockSpec((1, 8, _D), lambda p, i: (p, 0, 0)),
        out_shape=jax.ShapeDtypeStruct((2, 8, _D), jnp.float32),
    )(x, Wa, ba2, qa2)


def _comb_body(x_ref, acc_ref, o_ref):
    w0 = acc_ref[0, 0, 0] * (1.0 / _N)
    w1 = acc_ref[1, 0, 0] * (1.0 / _N)
    m = jnp.maximum(w0, w1)
    e0 = jnp.exp(w0 - m)
    e1 = jnp.exp(w1 - m)
    r = 1.0 / (e0 + e1)
    o_ref[...] = (e0 * r) * x_ref[0] + (e1 * r) * x_ref[1]


def _comb_call(x, wa):
    return pl.pallas_call(
        _comb_body,
        grid=(_N // _BN,),
        in_specs=[
            pl.BlockSpec((2, _BN, _D), lambda i: (0, i, 0)),
            pl.BlockSpec((2, 8, _D), lambda i: (0, 0, 0)),
        ],
        out_specs=pl.BlockSpec((_BN, _D), lambda i: (i, 0)),
        out_shape=jax.ShapeDtypeStruct((_N, _D), jnp.float32),
    )(x, wa)


def kernel(h, edge_index_0, edge_index_1, W_hidden, b_hidden, W_att, b_att,
           q_att):
    edges = jnp.stack([edge_index_0, edge_index_1]).reshape(4 * _E)
    hist = _deg_kernel(edges)
    z, xs, ns, nd = _prep_call(h, W_hidden, b_hidden.reshape(1, _D), hist)
    x = None
    for k in range(3):
        agg = _agg_kernel(xs.reshape(2 * _NP, _D), edges)
        if k < 2:
            xs = _upd_call(agg, z, ns, nd)
        else:
            x = _fin_call(agg, z, nd)
    wa = _att_call(x, W_att, b_att.reshape(1, _D), q_att.reshape(1, _D))
    return _comb_call(x, wa)


# speedup vs baseline: 1.2478x; 1.1853x over previous
import functools

import jax
import jax.numpy as jnp
from jax import lax
from jax.experimental import pallas as pl
from jax.experimental.pallas import tpu as pltpu
import jax.experimental.pallas.tpu_sc as plsc

_N = 10000
_NP = 10240
_E = 320000
_D = 128
_ALPHA = 0.1
_BETA = 1.0 - _ALPHA

_NC = 2
_NS = 16
_CH = 80
_EW = _E // _NS
_NCH = _EW // _CH
_RW = _NP // _NS
_ZR = 128
_NB = 2

_mesh = plsc.VectorSubcoreMesh(
    core_axis_name="c", subcore_axis_name="s", num_cores=_NC, num_subcores=_NS)


@functools.partial(
    pl.kernel,
    out_type=jax.ShapeDtypeStruct((2, _NP, _D), jnp.float32),
    mesh=_mesh,
    scratch_types=[
        pltpu.VMEM_SHARED((_NP, _D), jnp.float32),
        pltpu.VMEM((_ZR, _D), jnp.float32),
        pltpu.VMEM((_CH, _D), jnp.float32),
        pltpu.VMEM((_CH,), jnp.int32),
        pltpu.VMEM((_CH,), jnp.int32),
    ],
)
def _deg_kernel(edges, hist_out, acc, zbuf, ones_s, idx_s, idx_d):
    c = lax.axis_index("c")
    s = lax.axis_index("s")
    zero16 = jnp.zeros((16,), jnp.float32)
    one16 = jnp.ones((16,), jnp.float32)

    def _fill_o(i, carry):
        for u in range(_D // 16):
            ones_s[i, pl.ds(u * 16, 16)] = one16 if u < 4 else zero16
        return carry

    lax.fori_loop(0, _CH, _fill_o, 0)

    def _fill_z(i, carry):
        for u in range(_D // 16):
            zbuf[i, pl.ds(u * 16, 16)] = zero16
        return carry

    lax.fori_loop(0, _ZR, _fill_z, 0)
    for q in range(_RW // _ZR):
        pltpu.sync_copy(zbuf, acc.at[pl.ds(s * _RW + q * _ZR, _ZR)])

    def _fill_d(i, carry):
        for u in range(_D // 16):
            zbuf[i, pl.ds(u * 16, 16)] = zero16 if u < 4 else one16
        return carry

    lax.fori_loop(0, _CH, _fill_d, 0)
    plsc.subcore_barrier()

    def _chunk(j, carry):
        off = s * _EW + j * _CH
        pltpu.sync_copy(edges.at[pl.ds((c * 2 + 0) * _E + off, _CH)], idx_s)
        pltpu.sync_copy(edges.at[pl.ds((c * 2 + 1) * _E + off, _CH)], idx_d)
        pltpu.sync_copy(ones_s, acc.at[idx_s], add=True)
        pltpu.sync_copy(zbuf.at[pl.ds(0, _CH)], acc.at[idx_d], add=True)
        return carry

    lax.fori_loop(0, _NCH, _chunk, 0)
    plsc.subcore_barrier()

    for q in range(_RW // _ZR):
        r0 = s * _RW + q * _ZR
        pltpu.sync_copy(acc.at[pl.ds(r0, _ZR)], zbuf)
        pltpu.sync_copy(zbuf, hist_out.at[c, pl.ds(r0, _ZR)])


_SB = 10
_NSB = _NCH // _SB


@functools.partial(
    pl.kernel,
    out_type=jax.ShapeDtypeStruct((2, _NP, _D), jnp.float32),
    mesh=_mesh,
    scratch_types=[
        pltpu.VMEM_SHARED((_NP, _D), jnp.float32),
        [pltpu.VMEM((_CH, _D), jnp.float32)] * _NB,
        pltpu.VMEM((_SB * _CH,), jnp.int32),
        pltpu.VMEM((_SB * _CH,), jnp.int32),
        [pltpu.SemaphoreType.DMA] * _NB,
    ],
)
def _agg_kernel(xs, edges, agg_out, acc, rows, isb, idb, gsem):
    c = lax.axis_index("c")
    s = lax.axis_index("s")
    zero16 = jnp.zeros((16,), jnp.float32)
    cbase = c * _NP

    def _zrow(i, carry):
        for u in range(_D // 16):
            rows[0][i, pl.ds(u * 16, 16)] = zero16
        return carry

    lax.fori_loop(0, _CH, _zrow, 0)
    for q in range(_RW // _CH):
        pltpu.sync_copy(rows[0], acc.at[pl.ds(s * _RW + q * _CH, _CH)])
    plsc.subcore_barrier()

    def _sblock(b, carry):
        off = s * _EW + b * (_SB * _CH)
        pltpu.sync_copy(edges.at[pl.ds((c * 2 + 0) * _E + off, _SB * _CH)],
                        isb)
        pltpu.sync_copy(edges.at[pl.ds((c * 2 + 1) * _E + off, _SB * _CH)],
                        idb)
        for u in range(_SB * _CH // 16):
            isb[pl.ds(u * 16, 16)] = isb[pl.ds(u * 16, 16)] + cbase

        def _gather(k, t):
            pltpu.async_copy(xs.at[isb.at[pl.ds(k * _CH, _CH)]], rows[t],
                             gsem[t])

        def _gwait(k, t):
            pltpu.make_async_copy(xs.at[isb.at[pl.ds(k * _CH, _CH)]],
                                  rows[t], gsem[t]).wait()

        def _scat(k, t):
            pltpu.sync_copy(rows[t], acc.at[idb.at[pl.ds(k * _CH, _CH)]],
                            add=True)

        for t in range(_NB):
            _gather(t, t)
        for k in range(_SB // _NB - 1):
            base = k * _NB
            for t in range(_NB):
                _gwait(base + t, t)
                _scat(base + t, t)
                _gather(base + _NB + t, t)
        base = _SB - _NB
        for t in range(_NB):
            _gwait(base + t, t)
            _scat(base + t, t)
        return carry

    lax.fori_loop(0, _NSB, _sblock, 0)
    plsc.subcore_barrier()

    for q in range(_RW // _CH):
        r0 = s * _RW + q * _CH
        pltpu.sync_copy(acc.at[pl.ds(r0, _CH)], rows[0])
        pltpu.sync_copy(rows[0], agg_out.at[c, pl.ds(r0, _CH)])


_BN = 1000


def _prep_body(h_ref, w_ref, b_ref, hist_ref, z_ref, xs_ref, ns_ref, nd_ref):
    zb = jnp.maximum(
        jnp.dot(h_ref[...], w_ref[...], preferred_element_type=jnp.float32)
        + b_ref[...], 0.0)
    z_ref[...] = zb
    h3 = hist_ref[...]
    ds_ = h3[:, :, 0]
    dd_ = h3[:, :, 64]
    ns = jnp.where(ds_ > 0, lax.rsqrt(ds_), 0.0)[:, :, None]
    ns_ref[...] = ns
    nd_ref[...] = jnp.where(dd_ > 0, lax.rsqrt(dd_), 0.0)[:, :, None]
    xs_ref[...] = zb[None, :, :] * ns


def _prep_call(h, W, b2, hist):
    return pl.pallas_call(
        _prep_body,
        grid=(_N // _BN,),
        in_specs=[
            pl.BlockSpec((_BN, _D), lambda i: (i, 0)),
            pl.BlockSpec((_D, _D), lambda i: (0, 0)),
            pl.BlockSpec((1, _D), lambda i: (0, 0)),
            pl.BlockSpec((2, _BN, _D), lambda i: (0, i, 0)),
        ],
        out_specs=[
            pl.BlockSpec((_BN, _D), lambda i: (i, 0)),
            pl.BlockSpec((2, _BN, _D), lambda i: (0, i, 0)),
            pl.BlockSpec((2, _BN, 1), lambda i: (0, i, 0)),
            pl.BlockSpec((2, _BN, 1), lambda i: (0, i, 0)),
        ],
        out_shape=[
            jax.ShapeDtypeStruct((_N, _D), jnp.float32),
            jax.ShapeDtypeStruct((2, _NP, _D), jnp.float32),
            jax.ShapeDtypeStruct((2, _N, 1), jnp.float32),
            jax.ShapeDtypeStruct((2, _N, 1), jnp.float32),
        ],
    )(h, W, b2, hist)


def _upd_body(agg_ref, z_ref, ns_ref, nd_ref, xs_ref):
    ns = ns_ref[0]
    nd = nd_ref[0]
    xs_ref[0] = (_BETA * ns * nd) * agg_ref[0] + (_ALPHA * ns) * z_ref[...]


def _upd_call(agg, z, ns, nd):
    return pl.pallas_call(
        _upd_body,
        grid=(2, _N // _BN),
        in_specs=[
            pl.BlockSpec((1, _BN, _D), lambda p, i: (p, i, 0)),
            pl.BlockSpec((_BN, _D), lambda p, i: (i, 0)),
            pl.BlockSpec((1, _BN, 1), lambda p, i: (p, i, 0)),
            pl.BlockSpec((1, _BN, 1), lambda p, i: (p, i, 0)),
        ],
        out_specs=pl.BlockSpec((1, _BN, _D), lambda p, i: (p, i, 0)),
        out_shape=jax.ShapeDtypeStruct((2, _NP, _D), jnp.float32),
    )(agg, z, ns, nd)


def _fin_body(agg_ref, z_ref, nd_ref, x_ref):
    nd = nd_ref[0]
    x_ref[0] = (_BETA * nd) * agg_ref[0] + _ALPHA * z_ref[...]


def _fin_call(agg, z, nd):
    return pl.pallas_call(
        _fin_body,
        grid=(2, _N // _BN),
        in_specs=[
            pl.BlockSpec((1, _BN, _D), lambda p, i: (p, i, 0)),
            pl.BlockSpec((_BN, _D), lambda p, i: (i, 0)),
            pl.BlockSpec((1, _BN, 1), lambda p, i: (p, i, 0)),
        ],
        out_specs=pl.BlockSpec((1, _BN, _D), lambda p, i: (p, i, 0)),
        out_shape=jax.ShapeDtypeStruct((2, _N, _D), jnp.float32),
    )(agg, z, nd)


def _att_body(x_ref, wa_ref, ba_ref, qa_ref, acc_ref):
    i = pl.program_id(1)

    @pl.when(i == 0)
    def _():
        acc_ref[...] = jnp.zeros_like(acc_ref)

    t = jnp.tanh(
        jnp.dot(x_ref[0], wa_ref[...], preferred_element_type=jnp.float32)
        + ba_ref[...])
    acc_ref[...] += jnp.sum(t * qa_ref[...])


def _att_call(x, Wa, ba2, qa2):
    return pl.pallas_call(
        _att_body,
        grid=(2, _N // _BN),
        in_specs=[
            pl.BlockSpec((1, _BN, _D), lambda p, i: (p, i, 0)),
            pl.BlockSpec((_D, _D), lambda p, i: (0, 0)),
            pl.BlockSpec((1, _D), lambda p, i: (0, 0)),
            pl.BlockSpec((1, _D), lambda p, i: (0, 0)),
        ],
        out_specs=pl.BlockSpec((1, 8, _D), lambda p, i: (p, 0, 0)),
        out_shape=jax.ShapeDtypeStruct((2, 8, _D), jnp.float32),
    )(x, Wa, ba2, qa2)


def _comb_body(x_ref, acc_ref, o_ref):
    w0 = acc_ref[0, 0, 0] * (1.0 / _N)
    w1 = acc_ref[1, 0, 0] * (1.0 / _N)
    m = jnp.maximum(w0, w1)
    e0 = jnp.exp(w0 - m)
    e1 = jnp.exp(w1 - m)
    r = 1.0 / (e0 + e1)
    o_ref[...] = (e0 * r) * x_ref[0] + (e1 * r) * x_ref[1]


def _comb_call(x, wa):
    return pl.pallas_call(
        _comb_body,
        grid=(_N // _BN,),
        in_specs=[
            pl.BlockSpec((2, _BN, _D), lambda i: (0, i, 0)),
            pl.BlockSpec((2, 8, _D), lambda i: (0, 0, 0)),
        ],
        out_specs=pl.BlockSpec((_BN, _D), lambda i: (i, 0)),
        out_shape=jax.ShapeDtypeStruct((_N, _D), jnp.float32),
    )(x, wa)


def kernel(h, edge_index_0, edge_index_1, W_hidden, b_hidden, W_att, b_att,
           q_att):
    edges = jnp.stack([edge_index_0, edge_index_1]).reshape(4 * _E)
    hist = _deg_kernel(edges)
    z, xs, ns, nd = _prep_call(h, W_hidden, b_hidden.reshape(1, _D), hist)
    x = None
    for k in range(3):
        agg = _agg_kernel(xs.reshape(2 * _NP, _D), edges)
        if k < 2:
            xs = _upd_call(agg, z, ns, nd)
        else:
            x = _fin_call(agg, z, nd)
    wa = _att_call(x, W_att, b_att.reshape(1, _D), q_att.reshape(1, _D))
    return _comb_call(x, wa)


# speedup vs baseline: 1.4386x; 1.1528x over previous
import functools

import jax
import jax.numpy as jnp
from jax import lax
from jax.experimental import pallas as pl
from jax.experimental.pallas import tpu as pltpu
import jax.experimental.pallas.tpu_sc as plsc

_N = 10000
_NP = 10240
_E = 320000
_D = 128
_ALPHA = 0.1
_BETA = 1.0 - _ALPHA

_NC = 2
_NS = 16
_CH = 80
_EW = _E // _NS
_NCH = _EW // _CH
_RW = _NP // _NS
_ZR = 128
_NB = 2
_SB = 10
_NSB = _NCH // _SB

_mesh = plsc.VectorSubcoreMesh(
    core_axis_name="c", subcore_axis_name="s", num_cores=_NC, num_subcores=_NS)


@functools.partial(
    pl.kernel,
    out_type=jax.ShapeDtypeStruct((2, _NP, _D), jnp.float32),
    mesh=_mesh,
    scratch_types=[
        pltpu.VMEM_SHARED((_NP, _D), jnp.float32),
        pltpu.VMEM((_ZR, _D), jnp.float32),
        pltpu.VMEM((_CH, _D), jnp.float32),
        pltpu.VMEM((_SB * _CH,), jnp.int32),
        pltpu.VMEM((_SB * _CH,), jnp.int32),
    ],
)
def _deg_kernel(edges, hist_out, acc, zbuf, ones_s, isb, idb):
    c = lax.axis_index("c")
    s = lax.axis_index("s")
    zero16 = jnp.zeros((16,), jnp.float32)
    one16 = jnp.ones((16,), jnp.float32)

    def _fill_o(i, carry):
        for u in range(_D // 16):
            ones_s[i, pl.ds(u * 16, 16)] = one16 if u < 4 else zero16
        return carry

    lax.fori_loop(0, _CH, _fill_o, 0)

    def _fill_z(i, carry):
        for u in range(_D // 16):
            zbuf[i, pl.ds(u * 16, 16)] = zero16
        return carry

    lax.fori_loop(0, _ZR, _fill_z, 0)
    for q in range(_RW // _ZR):
        pltpu.sync_copy(zbuf, acc.at[pl.ds(s * _RW + q * _ZR, _ZR)])

    def _fill_d(i, carry):
        for u in range(_D // 16):
            zbuf[i, pl.ds(u * 16, 16)] = zero16 if u < 4 else one16
        return carry

    lax.fori_loop(0, _CH, _fill_d, 0)
    plsc.subcore_barrier()

    def _sblock(b, carry):
        off = s * _EW + b * (_SB * _CH)
        pltpu.sync_copy(edges.at[pl.ds((c * 2 + 0) * _E + off, _SB * _CH)],
                        isb)
        pltpu.sync_copy(edges.at[pl.ds((c * 2 + 1) * _E + off, _SB * _CH)],
                        idb)
        for k in range(_SB):
            pltpu.sync_copy(ones_s, acc.at[isb.at[pl.ds(k * _CH, _CH)]],
                            add=True)
            pltpu.sync_copy(zbuf.at[pl.ds(0, _CH)],
                            acc.at[idb.at[pl.ds(k * _CH, _CH)]], add=True)
        return carry

    lax.fori_loop(0, _NSB, _sblock, 0)
    plsc.subcore_barrier()

    for q in range(_RW // _ZR):
        r0 = s * _RW + q * _ZR
        pltpu.sync_copy(acc.at[pl.ds(r0, _ZR)], zbuf)
        pltpu.sync_copy(zbuf, hist_out.at[c, pl.ds(r0, _ZR)])


@functools.partial(
    pl.kernel,
    out_type=jax.ShapeDtypeStruct((2, _NP, _D), jnp.float32),
    mesh=_mesh,
    scratch_types=[
        pltpu.VMEM_SHARED((_NP, _D), jnp.float32),
        [pltpu.VMEM((_CH, _D), jnp.float32)] * _NB,
        pltpu.VMEM((_SB * _CH,), jnp.int32),
        pltpu.VMEM((_SB * _CH,), jnp.int32),
        [pltpu.SemaphoreType.DMA] * _NB,
    ],
)
def _agg_kernel(xs, edges, agg_out, acc, rows, isb, idb, gsem):
    c = lax.axis_index("c")
    s = lax.axis_index("s")
    zero16 = jnp.zeros((16,), jnp.float32)
    cbase = c * _NP

    def _zrow(i, carry):
        for u in range(_D // 16):
            rows[0][i, pl.ds(u * 16, 16)] = zero16
        return carry

    lax.fori_loop(0, _CH, _zrow, 0)
    for q in range(_RW // _CH):
        pltpu.sync_copy(rows[0], acc.at[pl.ds(s * _RW + q * _CH, _CH)])
    plsc.subcore_barrier()

    def _sblock(b, carry):
        off = s * _EW + b * (_SB * _CH)
        pltpu.sync_copy(edges.at[pl.ds((c * 2 + 0) * _E + off, _SB * _CH)],
                        isb)
        pltpu.sync_copy(edges.at[pl.ds((c * 2 + 1) * _E + off, _SB * _CH)],
                        idb)
        for u in range(_SB * _CH // 16):
            isb[pl.ds(u * 16, 16)] = isb[pl.ds(u * 16, 16)] + cbase

        def _gather(k, t):
            pltpu.async_copy(xs.at[isb.at[pl.ds(k * _CH, _CH)]], rows[t],
                             gsem[t])

        def _gwait(k, t):
            pltpu.make_async_copy(xs.at[isb.at[pl.ds(k * _CH, _CH)]],
                                  rows[t], gsem[t]).wait()

        def _scat(k, t):
            pltpu.sync_copy(rows[t], acc.at[idb.at[pl.ds(k * _CH, _CH)]],
                            add=True)

        for t in range(_NB):
            _gather(t, t)
        for k in range(_SB // _NB - 1):
            base = k * _NB
            for t in range(_NB):
                _gwait(base + t, t)
                _scat(base + t, t)
                _gather(base + _NB + t, t)
        base = _SB - _NB
        for t in range(_NB):
            _gwait(base + t, t)
            _scat(base + t, t)
        return carry

    lax.fori_loop(0, _NSB, _sblock, 0)
    plsc.subcore_barrier()

    for q in range(_RW // _CH):
        r0 = s * _RW + q * _CH
        pltpu.sync_copy(acc.at[pl.ds(r0, _CH)], rows[0])
        pltpu.sync_copy(rows[0], agg_out.at[c, pl.ds(r0, _CH)])


_BN = 1000


def _prep_body(h_ref, w_ref, b_ref, hist_ref, z_ref, xs_ref, ns_ref, nd_ref):
    zb = jnp.maximum(
        jnp.dot(h_ref[...], w_ref[...], preferred_element_type=jnp.float32)
        + b_ref[...], 0.0)
    z_ref[...] = zb
    h3 = hist_ref[...]
    ds_ = h3[:, :, 0]
    dd_ = h3[:, :, 64]
    ns = jnp.where(ds_ > 0, lax.rsqrt(ds_), 0.0)[:, :, None]
    ns_ref[...] = ns
    nd_ref[...] = jnp.where(dd_ > 0, lax.rsqrt(dd_), 0.0)[:, :, None]
    xs_ref[...] = zb[None, :, :] * ns


def _prep_call(h, W, b2, hist):
    return pl.pallas_call(
        _prep_body,
        grid=(_N // _BN,),
        in_specs=[
            pl.BlockSpec((_BN, _D), lambda i: (i, 0)),
            pl.BlockSpec((_D, _D), lambda i: (0, 0)),
            pl.BlockSpec((1, _D), lambda i: (0, 0)),
            pl.BlockSpec((2, _BN, _D), lambda i: (0, i, 0)),
        ],
        out_specs=[
            pl.BlockSpec((_BN, _D), lambda i: (i, 0)),
            pl.BlockSpec((2, _BN, _D), lambda i: (0, i, 0)),
            pl.BlockSpec((2, _BN, 1), lambda i: (0, i, 0)),
            pl.BlockSpec((2, _BN, 1), lambda i: (0, i, 0)),
        ],
        out_shape=[
            jax.ShapeDtypeStruct((_N, _D), jnp.float32),
            jax.ShapeDtypeStruct((2, _NP, _D), jnp.float32),
            jax.ShapeDtypeStruct((2, _N, 1), jnp.float32),
            jax.ShapeDtypeStruct((2, _N, 1), jnp.float32),
        ],
    )(h, W, b2, hist)


def _upd_body(agg_ref, z_ref, ns_ref, nd_ref, xs_ref):
    ns = ns_ref[0]
    nd = nd_ref[0]
    xs_ref[0] = (_BETA * ns * nd) * agg_ref[0] + (_ALPHA * ns) * z_ref[...]


def _upd_call(agg, z, ns, nd):
    return pl.pallas_call(
        _upd_body,
        grid=(2, _N // _BN),
        in_specs=[
            pl.BlockSpec((1, _BN, _D), lambda p, i: (p, i, 0)),
            pl.BlockSpec((_BN, _D), lambda p, i: (i, 0)),
            pl.BlockSpec((1, _BN, 1), lambda p, i: (p, i, 0)),
            pl.BlockSpec((1, _BN, 1), lambda p, i: (p, i, 0)),
        ],
        out_specs=pl.BlockSpec((1, _BN, _D), lambda p, i: (p, i, 0)),
        out_shape=jax.ShapeDtypeStruct((2, _NP, _D), jnp.float32),
    )(agg, z, ns, nd)


def _fin_body(agg_ref, z_ref, nd_ref, x_ref):
    nd = nd_ref[0]
    x_ref[0] = (_BETA * nd) * agg_ref[0] + _ALPHA * z_ref[...]


def _fin_call(agg, z, nd):
    return pl.pallas_call(
        _fin_body,
        grid=(2, _N // _BN),
        in_specs=[
            pl.BlockSpec((1, _BN, _D), lambda p, i: (p, i, 0)),
            pl.BlockSpec((_BN, _D), lambda p, i: (i, 0)),
            pl.BlockSpec((1, _BN, 1), lambda p, i: (p, i, 0)),
        ],
        out_specs=pl.BlockSpec((1, _BN, _D), lambda p, i: (p, i, 0)),
        out_shape=jax.ShapeDtypeStruct((2, _N, _D), jnp.float32),
    )(agg, z, nd)


def _att_body(x_ref, wa_ref, ba_ref, qa_ref, acc_ref):
    i = pl.program_id(1)

    @pl.when(i == 0)
    def _():
        acc_ref[...] = jnp.zeros_like(acc_ref)

    t = jnp.tanh(
        jnp.dot(x_ref[0], wa_ref[...], preferred_element_type=jnp.float32)
        + ba_ref[...])
    acc_ref[...] += jnp.sum(t * qa_ref[...])


def _att_call(x, Wa, ba2, qa2):
    return pl.pallas_call(
        _att_body,
        grid=(2, _N // _BN),
        in_specs=[
            pl.BlockSpec((1, _BN, _D), lambda p, i: (p, i, 0)),
            pl.BlockSpec((_D, _D), lambda p, i: (0, 0)),
            pl.BlockSpec((1, _D), lambda p, i: (0, 0)),
            pl.BlockSpec((1, _D), lambda p, i: (0, 0)),
        ],
        out_specs=pl.BlockSpec((1, 8, _D), lambda p, i: (p, 0, 0)),
        out_shape=jax.ShapeDtypeStruct((2, 8, _D), jnp.float32),
    )(x, Wa, ba2, qa2)


def _comb_body(x_ref, acc_ref, o_ref):
    w0 = acc_ref[0, 0, 0] * (1.0 / _N)
    w1 = acc_ref[1, 0, 0] * (1.0 / _N)
    m = jnp.maximum(w0, w1)
    e0 = jnp.exp(w0 - m)
    e1 = jnp.exp(w1 - m)
    r = 1.0 / (e0 + e1)
    o_ref[...] = (e0 * r) * x_ref[0] + (e1 * r) * x_ref[1]


def _comb_call(x, wa):
    return pl.pallas_call(
        _comb_body,
        grid=(_N // _BN,),
        in_specs=[
            pl.BlockSpec((2, _BN, _D), lambda i: (0, i, 0)),
            pl.BlockSpec((2, 8, _D), lambda i: (0, 0, 0)),
        ],
        out_specs=pl.BlockSpec((_BN, _D), lambda i: (i, 0)),
        out_shape=jax.ShapeDtypeStruct((_N, _D), jnp.float32),
    )(x, wa)


def kernel(h, edge_index_0, edge_index_1, W_hidden, b_hidden, W_att, b_att,
           q_att):
    edges = jnp.stack([edge_index_0, edge_index_1]).reshape(4 * _E)
    hist = _deg_kernel(edges)
    z, xs, ns, nd = _prep_call(h, W_hidden, b_hidden.reshape(1, _D), hist)
    x = None
    for k in range(3):
        agg = _agg_kernel(xs.reshape(2 * _NP, _D), edges)
        if k < 2:
            xs = _upd_call(agg, z, ns, nd)
        else:
            x = _fin_call(agg, z, nd)
    wa = _att_call(x, W_att, b_att.reshape(1, _D), q_att.reshape(1, _D))
    return _comb_call(x, wa)
